# baseline (speedup 1.0000x reference)
import jax
import jax.numpy as jnp
from jax import lax
from jax.experimental import pallas as pl
from jax.experimental.pallas import tpu as pltpu
from jax.experimental.pallas import tpu_sc as plsc

_B = 16384
_D = 200
_NC = 2
_NS = 16
_NW = _NC * _NS
_BPW = _B // _NW
_M = 128
_C = 256


def _gather_chunk(table_hbm, idx_v, off, rows_v, sem):
    cps = [pltpu.async_copy(table_hbm.at[idx_v.at[pl.ds(off + j * _M, _M)]],
                            rows_v.at[pl.ds(j * _M, _M)], sem)
           for j in range(_C // _M)]
    for c in cps:
        c.wait()


def _gather_body(entp, relp, e1_idx_hbm, rel_idx_hbm, out_hbm,
                 idx_v, rows_v, sem):
    wid = lax.axis_index("s") * _NC + lax.axis_index("c")
    base = wid * _BPW

    for col, table, idx_hbm in ((0, entp, e1_idx_hbm), (256, relp, rel_idx_hbm)):
        pltpu.sync_copy(idx_hbm.at[pl.ds(base, _BPW)], idx_v)
        for c0 in range(0, _BPW, _C):
            _gather_chunk(table, idx_v, c0, rows_v, sem)
            pltpu.sync_copy(rows_v,
                            out_hbm.at[pl.ds(base + c0, _C), pl.ds(col, 256)])


def _gather(entp, relp, e1_idx, rel_idx):
    mesh = plsc.VectorSubcoreMesh(core_axis_name="c", subcore_axis_name="s")
    f = pl.kernel(
        _gather_body,
        mesh=mesh,
        out_type=jax.ShapeDtypeStruct((_B, 512), jnp.float32),
        scratch_types=[
            pltpu.VMEM((_BPW,), jnp.int32),
            pltpu.VMEM((_C, 256), jnp.float32),
            pltpu.SemaphoreType.DMA,
        ],
    )
    return f(entp, relp, e1_idx, rel_idx)


def kernel(ent_table, rel_table, e1_idx, rel_idx):
    entp = jnp.pad(ent_table, ((0, 0), (0, 56)))
    relp = jnp.pad(rel_table, ((0, 12), (0, 56)))
    out512 = _gather(entp, relp, e1_idx, rel_idx)
    out = jnp.concatenate([out512[:, :200], out512[:, 256:456]], axis=1)
    return out.reshape(_B, 1, 20, 20)


# speedup vs baseline: 1.7761x; 1.7761x over previous
import jax
import jax.numpy as jnp
from jax import lax
from jax.experimental import pallas as pl
from jax.experimental.pallas import tpu as pltpu
from jax.experimental.pallas import tpu_sc as plsc

_B = 16384
_D = 200
_NC = 2
_NS = 16
_NW = _NC * _NS
_BPW = _B // _NW
_K = 4
_M = _BPW // _K


def _gather_half(table_hbm, idx_v, rows_v, sem):
    cps = [pltpu.async_copy(table_hbm.at[idx_v.at[pl.ds(j * _M, _M)]],
                            rows_v.at[pl.ds(j * _M, _M)], sem)
           for j in range(_K)]
    for c in cps:
        c.wait()


def _gather_body(entA, entB, relA, relB, e1_idx_hbm, rel_idx_hbm, out_hbm,
                 idx_v, rows_v, sem):
    wid = lax.axis_index("s") * _NC + lax.axis_index("c")
    base = wid * _BPW

    pltpu.sync_copy(e1_idx_hbm.at[pl.ds(base, _BPW)], idx_v)
    for col, half in ((0, entA), (128, entB)):
        _gather_half(half, idx_v, rows_v, sem)
        pltpu.sync_copy(rows_v, out_hbm.at[pl.ds(base, _BPW), pl.ds(col, 128)])

    pltpu.sync_copy(rel_idx_hbm.at[pl.ds(base, _BPW)], idx_v)
    for col, half in ((256, relA), (384, relB)):
        _gather_half(half, idx_v, rows_v, sem)
        pltpu.sync_copy(rows_v, out_hbm.at[pl.ds(base, _BPW), pl.ds(col, 128)])


def _gather(entA, entB, relA, relB, e1_idx, rel_idx):
    mesh = plsc.VectorSubcoreMesh(core_axis_name="c", subcore_axis_name="s")
    f = pl.kernel(
        _gather_body,
        mesh=mesh,
        out_type=jax.ShapeDtypeStruct((_B, 512), jnp.float32),
        scratch_types=[
            pltpu.VMEM((_BPW,), jnp.int32),
            pltpu.VMEM((_BPW, 128), jnp.float32),
            pltpu.SemaphoreType.DMA,
        ],
    )
    return f(entA, entB, relA, relB, e1_idx, rel_idx)


def kernel(ent_table, rel_table, e1_idx, rel_idx):
    entA = ent_table[:, :128]
    entB = jnp.pad(ent_table[:, 128:], ((0, 0), (0, 56)))
    relp = jnp.pad(rel_table, ((0, 12), (0, 0)))
    relA = relp[:, :128]
    relB = jnp.pad(relp[:, 128:], ((0, 0), (0, 56)))
    out512 = _gather(entA, entB, relA, relB, e1_idx, rel_idx)
    out = jnp.concatenate([out512[:, :200], out512[:, 256:456]], axis=1)
    return out.reshape(_B, 1, 20, 20)
